# Initial kernel scaffold; baseline (speedup 1.0000x reference)
import jax
import jax.numpy as jnp
from jax.experimental import pallas as pl


def kernel(x_user, x_item, edge_index_r0, edge_index_r1, W0, attn_l0, attn_r0, bias0, W1, attn_l1, attn_r1, bias1, loop_weight, h_bias):
    raise NotImplementedError("write your pallas kernel here")


# speedup vs baseline: 8.7477x; 8.7477x over previous
import functools

import jax
import jax.numpy as jnp
from jax import lax
from jax.experimental import pallas as pl
from jax.experimental.pallas import tpu as pltpu
from jax.experimental.pallas import tpu_sc as plsc

N = 10000
D = 128
H = 4
DH = 32
E = 320000
DE = D + 16

NTILES = 16
ET = E // NTILES
C = 80
NCH = ET // C

PAD_N = 10240
ROWS_T = PAD_N // NTILES
RB = 64
NRB = ROWS_T // RB


_B = 2000
_NB = N // _B


def _prep_body(xs_ref, xd_ref, w_ref, al_ref, ar_ref, brel_ref, loop_ref,
               hb_ref, fs_ref, er_ref, init_ref):
    xs = xs_ref[...]
    xd = xd_ref[...]
    w = w_ref[0]
    wext = jnp.concatenate(
        [w, jnp.dot(w, al_ref[0], preferred_element_type=jnp.float32)], axis=1)
    fs_ref[...] = jnp.dot(xs, wext, preferred_element_type=jnp.float32)
    wr = jnp.dot(w, ar_ref[0], preferred_element_type=jnp.float32)
    er_ref[...] = jnp.dot(xd, wr, preferred_element_type=jnp.float32)
    init_ref[...] = (jnp.dot(xd, loop_ref[...], preferred_element_type=jnp.float32)
                     + hb_ref[...] + brel_ref[0])


def _prep(x_cat, w_cat, al_cat, ar_cat, brel, loop_weight, hb2):
    return pl.pallas_call(
        _prep_body,
        grid=(2, _NB),
        in_specs=[
            pl.BlockSpec((_B, D), lambda r, i: (r * _NB + i, 0)),
            pl.BlockSpec((_B, D), lambda r, i: ((1 - r) * _NB + i, 0)),
            pl.BlockSpec((1, D, D), lambda r, i: (r, 0, 0)),
            pl.BlockSpec((1, D, 16), lambda r, i: (r, 0, 0)),
            pl.BlockSpec((1, D, 16), lambda r, i: (r, 0, 0)),
            pl.BlockSpec((1, 1, D), lambda r, i: (r, 0, 0)),
            pl.BlockSpec((D, D), lambda r, i: (0, 0)),
            pl.BlockSpec((1, D), lambda r, i: (0, 0)),
        ],
        out_specs=[
            pl.BlockSpec((_B, DE), lambda r, i: (r * _NB + i, 0)),
            pl.BlockSpec((_B, 16), lambda r, i: (r * _NB + i, 0)),
            pl.BlockSpec((_B, D), lambda r, i: (r * _NB + i, 0)),
        ],
        out_shape=[
            jax.ShapeDtypeStruct((2 * N, DE), jnp.float32),
            jax.ShapeDtypeStruct((2 * N, 16), jnp.float32),
            jax.ShapeDtypeStruct((2 * N, D), jnp.float32),
        ],
    )(x_cat, x_cat, w_cat, al_cat, ar_cat, brel, loop_weight, hb2)


_mesh = plsc.VectorSubcoreMesh(core_axis_name="c", subcore_axis_name="s")


@functools.partial(
    pl.kernel,
    mesh=_mesh,
    compiler_params=pltpu.CompilerParams(
        needs_layout_passes=False, use_tc_tiling_on_sc=False),
    out_type=jax.ShapeDtypeStruct((2 * PAD_N, D), jnp.float32),
    scratch_types=[
        pltpu.VMEM_SHARED((PAD_N, DE), jnp.float32),
        pltpu.VMEM((C, DE), jnp.float32),
        pltpu.VMEM((C, 16), jnp.float32),
        pltpu.VMEM((C, 16), jnp.float32),
        pltpu.VMEM((C,), jnp.int32),
        pltpu.VMEM((C,), jnp.int32),
        pltpu.VMEM((C,), jnp.int32),
        pltpu.VMEM((RB, DE), jnp.float32),
        pltpu.VMEM((RB, D), jnp.float32),
        pltpu.SemaphoreType.DMA,
    ],
)
def _edge_kernel(fs_hbm, er_hbm, src_hbm, dst_hbm, init_hbm, out_hbm,
                 acc_sh, fsb, erb, eeb, srcv, dstv, dstg, accb, initb, sem):
    r = lax.axis_index("c")
    s = lax.axis_index("s")
    iota16 = lax.iota(jnp.int32, 16)
    zeros16 = jnp.zeros((16,), jnp.float32)

    def _zrow(i, _):
        for j in range(DE // 16):
            fsb[i, pl.ds(j * 16, 16)] = zeros16
        eeb[i % C, pl.ds(0, 16)] = zeros16
        return 0
    lax.fori_loop(0, C, _zrow, 0)

    base = s * ROWS_T
    for k in range(ROWS_T // C):
        pltpu.sync_copy(fsb, acc_sh.at[pl.ds(base + k * C, C)])

    plsc.subcore_barrier()

    tbase = r * E + s * ET
    rowoff = r * N

    def _chunk(ci, _):
        off = tbase + ci * C
        pltpu.sync_copy(src_hbm.at[pl.ds(off, C)], srcv)
        pltpu.sync_copy(dst_hbm.at[pl.ds(off, C)], dstv)

        def _adj(g, _):
            sl = pl.ds(g * 16, 16)
            srcv[sl] = srcv[sl] + rowoff
            dstg[sl] = dstv[sl] + rowoff
            return 0
        lax.fori_loop(0, C // 16, _adj, 0)

        cp1 = pltpu.async_copy(fs_hbm.at[srcv], fsb, sem)
        cp2 = pltpu.async_copy(er_hbm.at[dstg], erb, sem)
        cp1.wait()
        cp2.wait()

        def _grp(g, _):
            rows = g * 16 + iota16
            for h in range(H):
                hv = jnp.full((16,), h, jnp.int32)
                le = plsc.load_gather(fsb, [rows, jnp.full((16,), D + h, jnp.int32)])
                re = plsc.load_gather(erb, [rows, hv])
                e = le + re
                e = jnp.where(e > 0.0, e, 0.2 * e)
                plsc.store_scatter(eeb, [rows, hv], jnp.exp(e))
            return 0
        lax.fori_loop(0, C // 16, _grp, 0)

        def _srow(row, _):
            rvec = jnp.full((16,), row, jnp.int32)
            for h in range(H):
                eesp = plsc.load_gather(eeb, [rvec, jnp.full((16,), h, jnp.int32)])
                for jj in range(2):
                    sl = pl.ds((h * 2 + jj) * 16, 16)
                    fsb[row, sl] = fsb[row, sl] * eesp
            fsb[row, pl.ds(D, 16)] = eeb[row, pl.ds(0, 16)]
            return 0
        lax.fori_loop(0, C, _srow, 0)

        pltpu.sync_copy(fsb, acc_sh.at[dstv], add=True)
        return 0
    lax.fori_loop(0, NCH, _chunk, 0)

    plsc.subcore_barrier()

    outoff = (1 - r) * PAD_N
    for k in range(NRB):
        rb0 = base + k * RB
        pltpu.sync_copy(acc_sh.at[pl.ds(rb0, RB)], accb)
        pltpu.sync_copy(init_hbm.at[pl.ds(r * PAD_N + rb0, RB)], initb)

        def _frow(row, _):
            rvec = jnp.full((16,), row, jnp.int32)
            for h in range(H):
                dsp = plsc.load_gather(
                    accb, [rvec, jnp.full((16,), D + h, jnp.int32)])
                dsp = dsp + 1e-9
                for jj in range(2):
                    sl = pl.ds((h * 2 + jj) * 16, 16)
                    initb[row, sl] = accb[row, sl] / dsp + initb[row, sl]
            return 0
        lax.fori_loop(0, RB, _frow, 0)
        pltpu.sync_copy(initb, out_hbm.at[pl.ds(outoff + rb0, RB)])


def _expand_attn(a):
    return jnp.zeros((D, 16), jnp.float32).at[
        jnp.arange(D), jnp.arange(D) // DH].set(a.reshape(-1))


def kernel(x_user, x_item, edge_index_r0, edge_index_r1,
           W0, attn_l0, attn_r0, bias0,
           W1, attn_l1, attn_r1, bias1,
           loop_weight, h_bias):
    x_cat = jnp.concatenate([x_user, x_item], axis=0)
    w_cat = jnp.stack([W0, W1])
    al_cat = jnp.stack([_expand_attn(attn_l0), _expand_attn(attn_l1)])
    ar_cat = jnp.stack([_expand_attn(attn_r0), _expand_attn(attn_r1)])
    brel = jnp.stack([bias0, bias1])[:, None, :]
    hb2 = h_bias[None, :]
    src_cat = jnp.concatenate(
        [edge_index_r0[0], edge_index_r1[0]]).astype(jnp.int32)
    dst_cat = jnp.concatenate(
        [edge_index_r0[1], edge_index_r1[1]]).astype(jnp.int32)

    fs_ext, er16, init_cat = _prep(
        x_cat, w_cat, al_cat, ar_cat, brel, loop_weight, hb2)

    init_pad = jnp.pad(init_cat.reshape(2, N, D),
                       ((0, 0), (0, PAD_N - N), (0, 0))).reshape(2 * PAD_N, D)
    out_flat = _edge_kernel(fs_ext, er16, src_cat, dst_cat, init_pad)
    return out_flat.reshape(2, PAD_N, D)[:, :N, :]


# speedup vs baseline: 10.8659x; 1.2422x over previous
import functools

import jax
import jax.numpy as jnp
from jax import lax
from jax.experimental import pallas as pl
from jax.experimental.pallas import tpu as pltpu
from jax.experimental.pallas import tpu_sc as plsc

N = 10000
D = 128
H = 4
DH = 32
E = 320000
DE = D + 16

NTILES = 16
ET = E // NTILES
C = 80
NCH = ET // C

PAD_N = 10240
ROWS_T = PAD_N // NTILES
RB = 32
NRB = ROWS_T // RB


_B = 2000
_NB = N // _B


def _prep_body(xs_ref, xd_ref, w_ref, al_ref, ar_ref, brel_ref, loop_ref,
               hb_ref, fs_ref, er_ref, init_ref):
    xs = xs_ref[...]
    xd = xd_ref[...]
    w = w_ref[0]
    wext = jnp.concatenate(
        [w, jnp.dot(w, al_ref[0], preferred_element_type=jnp.float32)], axis=1)
    fs_ref[...] = jnp.dot(xs, wext, preferred_element_type=jnp.float32)
    wr = jnp.dot(w, ar_ref[0], preferred_element_type=jnp.float32)
    er_ref[...] = jnp.dot(xd, wr, preferred_element_type=jnp.float32)
    init_ref[...] = (jnp.dot(xd, loop_ref[...], preferred_element_type=jnp.float32)
                     + hb_ref[...] + brel_ref[0])


def _prep(x_cat, w_cat, al_cat, ar_cat, brel, loop_weight, hb2):
    return pl.pallas_call(
        _prep_body,
        grid=(2, _NB),
        in_specs=[
            pl.BlockSpec((_B, D), lambda r, i: (r * _NB + i, 0)),
            pl.BlockSpec((_B, D), lambda r, i: ((1 - r) * _NB + i, 0)),
            pl.BlockSpec((1, D, D), lambda r, i: (r, 0, 0)),
            pl.BlockSpec((1, D, 16), lambda r, i: (r, 0, 0)),
            pl.BlockSpec((1, D, 16), lambda r, i: (r, 0, 0)),
            pl.BlockSpec((1, 1, D), lambda r, i: (r, 0, 0)),
            pl.BlockSpec((D, D), lambda r, i: (0, 0)),
            pl.BlockSpec((1, D), lambda r, i: (0, 0)),
        ],
        out_specs=[
            pl.BlockSpec((_B, DE), lambda r, i: (r * _NB + i, 0)),
            pl.BlockSpec((_B, 16), lambda r, i: (r * _NB + i, 0)),
            pl.BlockSpec((_B, D), lambda r, i: (r * _NB + i, 0)),
        ],
        out_shape=[
            jax.ShapeDtypeStruct((2 * N, DE), jnp.float32),
            jax.ShapeDtypeStruct((2 * N, 16), jnp.float32),
            jax.ShapeDtypeStruct((2 * N, D), jnp.float32),
        ],
    )(x_cat, x_cat, w_cat, al_cat, ar_cat, brel, loop_weight, hb2)


_mesh = plsc.VectorSubcoreMesh(core_axis_name="c", subcore_axis_name="s")


@functools.partial(
    pl.kernel,
    mesh=_mesh,
    compiler_params=pltpu.CompilerParams(
        needs_layout_passes=False, use_tc_tiling_on_sc=False),
    out_type=jax.ShapeDtypeStruct((2 * PAD_N, D), jnp.float32),
    scratch_types=[
        pltpu.VMEM_SHARED((PAD_N, DE), jnp.float32),
        pltpu.VMEM((C, DE), jnp.float32),
        pltpu.VMEM((C, DE), jnp.float32),
        pltpu.VMEM((C, 16), jnp.float32),
        pltpu.VMEM((C, 16), jnp.float32),
        pltpu.VMEM((C, 16), jnp.float32),
        pltpu.VMEM((C,), jnp.int32),
        pltpu.VMEM((C,), jnp.int32),
        pltpu.VMEM((C,), jnp.int32),
        pltpu.VMEM((C,), jnp.int32),
        pltpu.VMEM((C,), jnp.int32),
        pltpu.VMEM((C,), jnp.int32),
        pltpu.VMEM((RB, DE), jnp.float32),
        pltpu.VMEM((RB, D), jnp.float32),
        pltpu.SemaphoreType.DMA,
        pltpu.SemaphoreType.DMA,
    ],
)
def _edge_kernel(fs_hbm, er_hbm, src_hbm, dst_hbm, init_hbm, out_hbm,
                 acc_sh, fsbA, fsbB, erbA, erbB, eeb, srcvA, srcvB,
                 dstvA, dstvB, dstgA, dstgB, accb, initb, semA, semB):
    r = lax.axis_index("c")
    s = lax.axis_index("s")
    iota16 = lax.iota(jnp.int32, 16)
    zeros16 = jnp.zeros((16,), jnp.float32)

    def _zrow(i, _):
        for j in range(DE // 16):
            fsbA[i, pl.ds(j * 16, 16)] = zeros16
        eeb[i % C, pl.ds(0, 16)] = zeros16
        return 0
    lax.fori_loop(0, C, _zrow, 0)

    base = s * ROWS_T
    for k in range(ROWS_T // C):
        pltpu.sync_copy(fsbA, acc_sh.at[pl.ds(base + k * C, C)])

    plsc.subcore_barrier()

    tbase = r * E + s * ET
    rowoff = r * N

    def _load_and_start(ci, sv, dv, dg, fb, eb, sem):
        off = tbase + ci * C
        pltpu.sync_copy(src_hbm.at[pl.ds(off, C)], sv)
        pltpu.sync_copy(dst_hbm.at[pl.ds(off, C)], dv)

        def _adj(g, _):
            sl = pl.ds(g * 16, 16)
            sv[sl] = sv[sl] + rowoff
            dg[sl] = dv[sl] + rowoff
            return 0
        lax.fori_loop(0, C // 16, _adj, 0)
        pltpu.async_copy(fs_hbm.at[sv], fb, sem)
        pltpu.async_copy(er_hbm.at[dg], eb, sem)

    def _drain(sv, dg, fb, eb, sem):
        pltpu.make_async_copy(fs_hbm.at[sv], fb, sem).wait()
        pltpu.make_async_copy(er_hbm.at[dg], eb, sem).wait()

    def _compute(sv, dv, dg, fb, eb, sem):
        _drain(sv, dg, fb, eb, sem)

        def _grp(g, _):
            rows = g * 16 + iota16
            for h in range(H):
                hv = jnp.full((16,), h, jnp.int32)
                le = plsc.load_gather(fb, [rows, jnp.full((16,), D + h, jnp.int32)])
                re = plsc.load_gather(eb, [rows, hv])
                e = le + re
                e = jnp.where(e > 0.0, e, 0.2 * e)
                plsc.store_scatter(eeb, [rows, hv], jnp.exp(e))
            return 0
        lax.fori_loop(0, C // 16, _grp, 0)

        def _srow(row, _):
            rvec = jnp.full((16,), row, jnp.int32)
            for h in range(H):
                eesp = plsc.load_gather(eeb, [rvec, jnp.full((16,), h, jnp.int32)])
                for jj in range(2):
                    sl = pl.ds((h * 2 + jj) * 16, 16)
                    fb[row, sl] = fb[row, sl] * eesp
            fb[row, pl.ds(D, 16)] = eeb[row, pl.ds(0, 16)]
            return 0
        lax.fori_loop(0, C, _srow, 0)

        pltpu.sync_copy(fb, acc_sh.at[dv], add=True)

    _load_and_start(0, srcvA, dstvA, dstgA, fsbA, erbA, semA)

    def _pair(i, _):
        c0 = 2 * i
        _load_and_start(c0 + 1, srcvB, dstvB, dstgB, fsbB, erbB, semB)
        _compute(srcvA, dstvA, dstgA, fsbA, erbA, semA)
        nxt = lax.select(c0 + 2 < NCH, c0 + 2, 0)
        _load_and_start(nxt, srcvA, dstvA, dstgA, fsbA, erbA, semA)
        _compute(srcvB, dstvB, dstgB, fsbB, erbB, semB)
        return 0
    lax.fori_loop(0, NCH // 2, _pair, 0)
    _drain(srcvA, dstgA, fsbA, erbA, semA)

    plsc.subcore_barrier()

    outoff = (1 - r) * PAD_N

    def _fix(k, _):
        rb0 = base + k * RB
        pltpu.sync_copy(acc_sh.at[pl.ds(rb0, RB)], accb)
        pltpu.sync_copy(init_hbm.at[pl.ds(r * PAD_N + rb0, RB)], initb)

        def _frow(row, _):
            rvec = jnp.full((16,), row, jnp.int32)
            for h in range(H):
                dsp = plsc.load_gather(
                    accb, [rvec, jnp.full((16,), D + h, jnp.int32)])
                dsp = dsp + 1e-9
                for jj in range(2):
                    sl = pl.ds((h * 2 + jj) * 16, 16)
                    initb[row, sl] = accb[row, sl] / dsp + initb[row, sl]
            return 0
        lax.fori_loop(0, RB, _frow, 0)
        pltpu.sync_copy(initb, out_hbm.at[pl.ds(outoff + rb0, RB)])
        return 0
    lax.fori_loop(0, NRB, _fix, 0)


def _expand_attn(a):
    return jnp.zeros((D, 16), jnp.float32).at[
        jnp.arange(D), jnp.arange(D) // DH].set(a.reshape(-1))


def kernel(x_user, x_item, edge_index_r0, edge_index_r1,
           W0, attn_l0, attn_r0, bias0,
           W1, attn_l1, attn_r1, bias1,
           loop_weight, h_bias):
    x_cat = jnp.concatenate([x_user, x_item], axis=0)
    w_cat = jnp.stack([W0, W1])
    al_cat = jnp.stack([_expand_attn(attn_l0), _expand_attn(attn_l1)])
    ar_cat = jnp.stack([_expand_attn(attn_r0), _expand_attn(attn_r1)])
    brel = jnp.stack([bias0, bias1])[:, None, :]
    hb2 = h_bias[None, :]
    src_cat = jnp.concatenate(
        [edge_index_r0[0], edge_index_r1[0]]).astype(jnp.int32)
    dst_cat = jnp.concatenate(
        [edge_index_r0[1], edge_index_r1[1]]).astype(jnp.int32)

    fs_ext, er16, init_cat = _prep(
        x_cat, w_cat, al_cat, ar_cat, brel, loop_weight, hb2)

    init_pad = jnp.pad(init_cat.reshape(2, N, D),
                       ((0, 0), (0, PAD_N - N), (0, 0))).reshape(2 * PAD_N, D)
    out_flat = _edge_kernel(fs_ext, er16, src_cat, dst_cat, init_pad)
    return out_flat.reshape(2, PAD_N, D)[:, :N, :]
